# baseline (speedup 1.0000x reference)
import functools

import jax
import jax.numpy as jnp
from jax.experimental import pallas as pl
from jax.experimental.pallas import tpu as pltpu

EPS_PROBE = 0.01
K_TOP = 8
RMS_EPS = 1e-6
NORM_EPS = 1e-6


def _tokens_kernel(x_ref, v_ref, vfull_ref, w_ref, that_ref, phat_ref,
                   rms_ref, *, rb):
    i = pl.program_id(1)

    @pl.when(i == 0)
    def _():
        vf = vfull_ref[0]
        rms_ref[0] = jnp.sqrt(jnp.mean(vf * vf) + RMS_EPS)

    x = x_ref[0]
    v = v_ref[0]
    w = w_ref[...]
    rms = rms_ref[0]
    xp = x + (EPS_PROBE / rms) * v
    wb = w.astype(jnp.bfloat16)
    t = jax.lax.dot_general(
        x.astype(jnp.bfloat16), wb, (((1,), (0,)), ((), ())),
        preferred_element_type=jnp.float32)
    p = jax.lax.dot_general(
        xp.astype(jnp.bfloat16), wb, (((1,), (0,)), ((), ())),
        preferred_element_type=jnp.float32)
    tn = jnp.sqrt(jnp.sum(t * t, axis=1, keepdims=True)) + NORM_EPS
    pn = jnp.sqrt(jnp.sum(p * p, axis=1, keepdims=True)) + NORM_EPS
    that_ref[0] = (t / tn).astype(jnp.bfloat16)
    phat_ref[0] = (p / pn).astype(jnp.bfloat16)


def _sim_loss_kernel(ta_ref, pa_ref, tf_ref, pf_ref, o_ref, *, rb, n):
    i = pl.program_id(1)
    a = ta_ref[0]
    ap = pa_ref[0]
    bt = tf_ref[0]
    bp = pf_ref[0]
    s_t = jax.lax.dot_general(
        a, bt, (((1,), (1,)), ((), ())),
        preferred_element_type=jnp.float32)
    s_p = jax.lax.dot_general(
        ap, bp, (((1,), (1,)), ((), ())),
        preferred_element_type=jnp.float32)
    row = jax.lax.broadcasted_iota(jnp.int32, (rb, n), 0) + i * rb
    col = jax.lax.broadcasted_iota(jnp.int32, (rb, n), 1)
    s_orig = jnp.where(col == row, -2.0, s_t)
    m = jnp.max(s_orig, axis=1, keepdims=True)
    for _ in range(K_TOP - 1):
        m = jnp.max(jnp.where(s_orig < m, s_orig, -2.0), axis=1, keepdims=True)
    d_sel = jnp.where(s_orig >= m, s_p - s_t, 0.0)
    partial = jnp.sum(d_sel * d_sel)

    b = pl.program_id(0)

    @pl.when(i == 0)
    def _():
        o_ref[b, 0] = partial

    @pl.when(i != 0)
    def _():
        o_ref[b, 0] += partial


@jax.jit
def kernel(x_t, v_pred, W):
    B, N, D = x_t.shape
    RB = 1024
    nb = N // RB
    that, phat = pl.pallas_call(
        functools.partial(_tokens_kernel, rb=RB),
        grid=(B, nb),
        in_specs=[
            pl.BlockSpec((1, RB, D), lambda b, i: (b, i, 0)),
            pl.BlockSpec((1, RB, D), lambda b, i: (b, i, 0)),
            pl.BlockSpec((1, N, D), lambda b, i: (b, 0, 0)),
            pl.BlockSpec((D, D), lambda b, i: (0, 0)),
        ],
        scratch_shapes=[pltpu.SMEM((1,), jnp.float32)],
        out_specs=[
            pl.BlockSpec((1, RB, D), lambda b, i: (b, i, 0)),
            pl.BlockSpec((1, RB, D), lambda b, i: (b, i, 0)),
        ],
        out_shape=[
            jax.ShapeDtypeStruct((B, N, D), jnp.bfloat16),
            jax.ShapeDtypeStruct((B, N, D), jnp.bfloat16),
        ],
    )(x_t, v_pred, v_pred, W)

    RB2 = 1024
    nb2 = N // RB2
    acc = pl.pallas_call(
        functools.partial(_sim_loss_kernel, rb=RB2, n=N),
        grid=(B, nb2),
        in_specs=[
            pl.BlockSpec((1, RB2, D), lambda b, i: (b, i, 0)),
            pl.BlockSpec((1, RB2, D), lambda b, i: (b, i, 0)),
            pl.BlockSpec((1, N, D), lambda b, i: (b, 0, 0)),
            pl.BlockSpec((1, N, D), lambda b, i: (b, 0, 0)),
        ],
        out_specs=pl.BlockSpec((B, 1), lambda b, i: (0, 0),
                               memory_space=pltpu.SMEM),
        out_shape=jax.ShapeDtypeStruct((B, 1), jnp.float32),
    )(that, phat, that, phat)

    mask_sum = jnp.float32(K_TOP * N)
    return acc[:, 0] / (mask_sum + 1e-6)


# speedup vs baseline: 1.0125x; 1.0125x over previous
import functools

import jax
import jax.numpy as jnp
from jax.experimental import pallas as pl
from jax.experimental.pallas import tpu as pltpu

EPS_PROBE = 0.01
K_TOP = 8
RMS_EPS = 1e-6
NORM_EPS = 1e-6


def _tokens_kernel(x_ref, v_ref, vfull_ref, w_ref, that_ref, phat_ref,
                   rms_ref, *, rb):
    i = pl.program_id(1)

    @pl.when(i == 0)
    def _():
        vf = vfull_ref[0]
        rms_ref[0] = jnp.sqrt(jnp.mean(vf * vf) + RMS_EPS)

    x = x_ref[0]
    v = v_ref[0]
    w = w_ref[...]
    rms = rms_ref[0]
    xp = x + (EPS_PROBE / rms) * v
    wb = w.astype(jnp.bfloat16)
    xx = jnp.concatenate(
        [x.astype(jnp.bfloat16), xp.astype(jnp.bfloat16)], axis=0)
    tp = jax.lax.dot_general(
        xx, wb, (((1,), (0,)), ((), ())),
        preferred_element_type=jnp.float32)
    t = tp[:rb]
    p = tp[rb:]
    tn = jnp.sqrt(jnp.sum(t * t, axis=1, keepdims=True)) + NORM_EPS
    pn = jnp.sqrt(jnp.sum(p * p, axis=1, keepdims=True)) + NORM_EPS
    that_ref[0] = (t / tn).astype(jnp.bfloat16)
    phat_ref[0] = (p / pn).astype(jnp.bfloat16)


def _sim_loss_kernel(ta_ref, pa_ref, tf_ref, pf_ref, o_ref, *, rb, n):
    i = pl.program_id(1)
    a = ta_ref[0]
    ap = pa_ref[0]
    bt = tf_ref[0]
    bp = pf_ref[0]
    s_t = jax.lax.dot_general(
        a, bt, (((1,), (1,)), ((), ())),
        preferred_element_type=jnp.float32)
    s_p = jax.lax.dot_general(
        ap, bp, (((1,), (1,)), ((), ())),
        preferred_element_type=jnp.float32)
    row = jax.lax.broadcasted_iota(jnp.int32, (rb, n), 0) + i * rb
    col = jax.lax.broadcasted_iota(jnp.int32, (rb, n), 1)
    s_orig = jnp.where(col == row, -2.0, s_t)
    m = jnp.max(s_orig, axis=1, keepdims=True)
    for _ in range(K_TOP - 1):
        m = jnp.max(jnp.where(s_orig < m, s_orig, -2.0), axis=1, keepdims=True)
    d_sel = jnp.where(s_orig >= m, s_p - s_t, 0.0)
    partial = jnp.sum(d_sel * d_sel)

    b = pl.program_id(0)

    @pl.when(i == 0)
    def _():
        o_ref[b, 0] = partial

    @pl.when(i != 0)
    def _():
        o_ref[b, 0] += partial


@jax.jit
def kernel(x_t, v_pred, W):
    B, N, D = x_t.shape
    RB = 512
    nb = N // RB
    that, phat = pl.pallas_call(
        functools.partial(_tokens_kernel, rb=RB),
        grid=(B, nb),
        in_specs=[
            pl.BlockSpec((1, RB, D), lambda b, i: (b, i, 0)),
            pl.BlockSpec((1, RB, D), lambda b, i: (b, i, 0)),
            pl.BlockSpec((1, N, D), lambda b, i: (b, 0, 0)),
            pl.BlockSpec((D, D), lambda b, i: (0, 0)),
        ],
        scratch_shapes=[pltpu.SMEM((1,), jnp.float32)],
        out_specs=[
            pl.BlockSpec((1, RB, D), lambda b, i: (b, i, 0)),
            pl.BlockSpec((1, RB, D), lambda b, i: (b, i, 0)),
        ],
        out_shape=[
            jax.ShapeDtypeStruct((B, N, D), jnp.bfloat16),
            jax.ShapeDtypeStruct((B, N, D), jnp.bfloat16),
        ],
    )(x_t, v_pred, v_pred, W)

    RB2 = 1024
    nb2 = N // RB2
    acc = pl.pallas_call(
        functools.partial(_sim_loss_kernel, rb=RB2, n=N),
        grid=(B, nb2),
        in_specs=[
            pl.BlockSpec((1, RB2, D), lambda b, i: (b, i, 0)),
            pl.BlockSpec((1, RB2, D), lambda b, i: (b, i, 0)),
            pl.BlockSpec((1, N, D), lambda b, i: (b, 0, 0)),
            pl.BlockSpec((1, N, D), lambda b, i: (b, 0, 0)),
        ],
        out_specs=pl.BlockSpec((B, 1), lambda b, i: (0, 0),
                               memory_space=pltpu.SMEM),
        out_shape=jax.ShapeDtypeStruct((B, 1), jnp.float32),
    )(that, phat, that, phat)

    mask_sum = jnp.float32(K_TOP * N)
    return acc[:, 0] / (mask_sum + 1e-6)
